# Initial kernel scaffold; baseline (speedup 1.0000x reference)
import jax
import jax.numpy as jnp
from jax.experimental import pallas as pl


def kernel(token_ids, weight):
    raise NotImplementedError("write your pallas kernel here")


# speedup vs baseline: 1.7977x; 1.7977x over previous
import functools

import jax
import jax.numpy as jnp
from jax import lax
from jax.experimental import pallas as pl
from jax.experimental.pallas import tpu as pltpu
from jax.experimental.pallas import tpu_sc as plsc

_NUM_TOKENS = 16384
_SEQ = 50
_DIM = 64
_B = _NUM_TOKENS * _SEQ
_NC = 2
_NS = 16
_NW = _NC * _NS
_BPW = _B // _NW
_CHUNK = 512
_G = 128
_GPC = _CHUNK // _G
_NCHUNKS = _BPW // _CHUNK


def _emb_body(ids_hbm, w_hbm, out_hbm, idx_v, rows_v, sem):
  wid = lax.axis_index("s") * _NC + lax.axis_index("c")
  base = wid * _BPW

  def chunk(i, carry):
    off = base + i * _CHUNK
    pltpu.sync_copy(ids_hbm.at[pl.ds(off, _CHUNK)], idx_v)
    copies = []
    for g in range(_GPC):
      copies.append(
          pltpu.async_copy(
              w_hbm.at[idx_v.at[pl.ds(g * _G, _G)]],
              rows_v.at[pl.ds(g * _G, _G)],
              sem,
          )
      )
    for c in copies:
      c.wait()
    pltpu.sync_copy(rows_v, out_hbm.at[pl.ds(off, _CHUNK)])
    return carry

  lax.fori_loop(0, _NCHUNKS, chunk, 0)


@jax.jit
def _emb(ids_flat, weight):
  mesh = plsc.VectorSubcoreMesh(
      core_axis_name="c", subcore_axis_name="s",
      num_cores=_NC, num_subcores=_NS,
  )
  f = functools.partial(
      pl.kernel,
      mesh=mesh,
      out_type=jax.ShapeDtypeStruct((_B, _DIM), jnp.float32),
      scratch_types=[
          pltpu.VMEM((_CHUNK,), jnp.int32),
          pltpu.VMEM((_CHUNK, _DIM), jnp.float32),
          pltpu.SemaphoreType.DMA,
      ],
      compiler_params=pltpu.CompilerParams(use_tc_tiling_on_sc=False),
  )(_emb_body)
  return f(ids_flat, weight)


def kernel(token_ids, weight):
  ids_flat = token_ids.reshape(_B).astype(jnp.int32)
  out = _emb(ids_flat, weight)
  return out.reshape(_NUM_TOKENS, _SEQ, _DIM)


# speedup vs baseline: 1.8737x; 1.0423x over previous
import functools

import jax
import jax.numpy as jnp
from jax import lax
from jax.experimental import pallas as pl
from jax.experimental.pallas import tpu as pltpu
from jax.experimental.pallas import tpu_sc as plsc

_NUM_TOKENS = 16384
_SEQ = 50
_DIM = 64
_B = _NUM_TOKENS * _SEQ
_NC = 2
_NS = 16
_NW = _NC * _NS
_BPW = _B // _NW
_CHUNK = 640
_G = 128
_GPC = _CHUNK // _G
_NCHUNKS = _BPW // _CHUNK


def _emb_body(ids_hbm, w_hbm, out_hbm, idx_v, rows_v,
              isem0, isem1, gsem0, gsem1, osem0, osem1):
  wid = lax.axis_index("s") * _NC + lax.axis_index("c")
  base = wid * _BPW
  isems = (isem0, isem1)
  gsems = (gsem0, gsem1)
  osems = (osem0, osem1)

  def idx_desc(i, b):
    off = base + i * _CHUNK
    return pltpu.make_async_copy(
        ids_hbm.at[pl.ds(off, _CHUNK)], idx_v.at[b], isems[b])

  def gather_desc(g, b):
    return pltpu.make_async_copy(
        w_hbm.at[idx_v.at[b, pl.ds(g * _G, _G)]],
        rows_v.at[b, pl.ds(g * _G, _G)],
        gsems[b])

  def out_desc(i, b):
    off = base + i * _CHUNK
    return pltpu.make_async_copy(
        rows_v.at[b], out_hbm.at[pl.ds(off, _CHUNK)], osems[b])

  idx_desc(0, 0).start()

  @pl.loop(0, _NCHUNKS, step=2)
  def _outer(i0):
    for b in range(2):
      i = i0 + b

      @pl.when(i + 1 < _NCHUNKS)
      def _():
        idx_desc(i + 1, 1 - b).start()

      idx_desc(i, b).wait()

      @pl.when(i >= 2)
      def _():
        out_desc(i - 2, b).wait()

      descs = [gather_desc(g, b) for g in range(_GPC)]
      for d in descs:
        d.start()
      for d in descs:
        d.wait()
      out_desc(i, b).start()

  out_desc(_NCHUNKS - 2, 0).wait()
  out_desc(_NCHUNKS - 1, 1).wait()


@jax.jit
def _emb(ids_flat, weight):
  mesh = plsc.VectorSubcoreMesh(
      core_axis_name="c", subcore_axis_name="s",
      num_cores=_NC, num_subcores=_NS,
  )
  f = functools.partial(
      pl.kernel,
      mesh=mesh,
      out_type=jax.ShapeDtypeStruct((_B, _DIM), jnp.float32),
      scratch_types=[
          pltpu.VMEM((2, _CHUNK), jnp.int32),
          pltpu.VMEM((2, _CHUNK, _DIM), jnp.float32),
          pltpu.SemaphoreType.DMA,
          pltpu.SemaphoreType.DMA,
          pltpu.SemaphoreType.DMA,
          pltpu.SemaphoreType.DMA,
          pltpu.SemaphoreType.DMA,
          pltpu.SemaphoreType.DMA,
      ],
      compiler_params=pltpu.CompilerParams(use_tc_tiling_on_sc=False),
  )(_emb_body)
  return f(ids_flat, weight)


def kernel(token_ids, weight):
  ids_flat = token_ids.reshape(_B).astype(jnp.int32)
  out = _emb(ids_flat, weight)
  return out.reshape(_NUM_TOKENS, _SEQ, _DIM)


# speedup vs baseline: 1.8778x; 1.0022x over previous
import functools

import jax
import jax.numpy as jnp
from jax import lax
from jax.experimental import pallas as pl
from jax.experimental.pallas import tpu as pltpu
from jax.experimental.pallas import tpu_sc as plsc

_NUM_TOKENS = 16384
_SEQ = 50
_DIM = 64
_B = _NUM_TOKENS * _SEQ
_NC = 2
_NS = 16
_NW = _NC * _NS
_BPW = _B // _NW
_NBUF = 4
_CHUNK = 400
_G = 80
_GPC = _CHUNK // _G
_NCHUNKS = _BPW // _CHUNK


def _emb_body(ids_hbm, w_hbm, out_hbm, idx_v, rows_v, isems, gsems, osems):
  wid = lax.axis_index("s") * _NC + lax.axis_index("c")
  base = wid * _BPW

  def idx_desc(i, b):
    off = base + i * _CHUNK
    return pltpu.make_async_copy(
        ids_hbm.at[pl.ds(off, _CHUNK)], idx_v.at[b], isems[b])

  def gather_descs(b):
    return [
        pltpu.make_async_copy(
            w_hbm.at[idx_v.at[b, pl.ds(g * _G, _G)]],
            rows_v.at[b, pl.ds(g * _G, _G)],
            gsems[b])
        for g in range(_GPC)
    ]

  def out_desc(i, b):
    off = base + i * _CHUNK
    return pltpu.make_async_copy(
        rows_v.at[b], out_hbm.at[pl.ds(off, _CHUNK)], osems[b])

  idx_desc(0, 0).start()

  @pl.loop(0, _NCHUNKS, step=_NBUF)
  def _outer(i0):
    for b in range(_NBUF):
      i = i0 + b

      @pl.when(i + 1 < _NCHUNKS)
      def _():
        idx_desc(i + 1, (b + 1) % _NBUF).start()

      idx_desc(i, b).wait()

      @pl.when(i >= _NBUF)
      def _():
        out_desc(i - _NBUF, b).wait()

      for d in gather_descs(b):
        d.start()

      pb = (b + _NBUF - 1) % _NBUF

      @pl.when(i >= 1)
      def _():
        for d in gather_descs(pb):
          d.wait()
        out_desc(i - 1, pb).start()

  last = _NBUF - 1
  for d in gather_descs(last):
    d.wait()
  out_desc(_NCHUNKS - 1, last).start()
  for k in range(_NBUF):
    out_desc(_NCHUNKS - _NBUF + k, k).wait()


@jax.jit
def _emb(ids_flat, weight):
  mesh = plsc.VectorSubcoreMesh(
      core_axis_name="c", subcore_axis_name="s",
      num_cores=_NC, num_subcores=_NS,
  )
  f = functools.partial(
      pl.kernel,
      mesh=mesh,
      out_type=jax.ShapeDtypeStruct((_B, _DIM), jnp.float32),
      scratch_types=[
          pltpu.VMEM((_NBUF, _CHUNK), jnp.int32),
          pltpu.VMEM((_NBUF, _CHUNK, _DIM), jnp.float32),
          [pltpu.SemaphoreType.DMA] * _NBUF,
          [pltpu.SemaphoreType.DMA] * _NBUF,
          [pltpu.SemaphoreType.DMA] * _NBUF,
      ],
      compiler_params=pltpu.CompilerParams(use_tc_tiling_on_sc=False),
  )(_emb_body)
  return f(ids_flat, weight)


def kernel(token_ids, weight):
  ids_flat = token_ids.reshape(_B).astype(jnp.int32)
  out = _emb(ids_flat, weight)
  return out.reshape(_NUM_TOKENS, _SEQ, _DIM)
